# baseline (speedup 1.0000x reference)
import math

import jax
import jax.numpy as jnp
import numpy as np
from jax.experimental import pallas as pl
from jax.experimental.pallas import tpu as pltpu

D_MODEL = 512
MAX_LEN = 512


def _pe_table_ext() -> np.ndarray:
    pe = np.zeros((MAX_LEN, D_MODEL), dtype=np.float32)
    position = np.arange(0, MAX_LEN, dtype=np.float32)[:, None]
    div_term = np.exp(
        np.arange(0, D_MODEL, 2, dtype=np.float32) * -(math.log(10000.0) / D_MODEL)
    )
    pe[:, 0::2] = np.sin(position * div_term)
    pe[:, 1::2] = np.cos(position * div_term)
    return np.concatenate([np.zeros((1, D_MODEL), np.float32), pe], axis=0)


def _gather_indices(t_lens, D) -> np.ndarray:
    parts = []
    for t in t_lens:
        parts.append(np.zeros((1,), np.int32))
        parts.append(np.linspace(0, D - 1, t).astype(np.int32) + 1)
    return np.concatenate(parts)


def kernel(modal_feat_0, modal_feat_1, modal_feat_2):
    modal_feats = (modal_feat_0, modal_feat_1, modal_feat_2)
    batch = modal_feats[0].shape[0]
    D = modal_feats[0].shape[1] - 1
    t_lens = [m.shape[1] - 1 for m in modal_feats]
    seq = sum(t_lens) + len(t_lens)

    table = _pe_table_ext()
    idx = _gather_indices(t_lens, D)
    nrows = table.shape[0]
    onehot = np.zeros((seq, nrows), np.float32)
    onehot[np.arange(seq), idx] = 1.0

    rows4, lanes = seq * 4, D_MODEL // 4

    def body(oh_ref, tab_ref, o_ref, temp, sem):
        temp[...] = jnp.dot(
            oh_ref[...], tab_ref[...], preferred_element_type=jnp.float32
        ).reshape(rows4, lanes)
        copies = [
            pltpu.make_async_copy(temp, o_ref.at[b], sem)
            for b in range(batch)
        ]
        for c in copies:
            c.start()
        for c in copies:
            c.wait()

    out2d = pl.pallas_call(
        body,
        in_specs=[
            pl.BlockSpec((seq, nrows), lambda: (0, 0)),
            pl.BlockSpec((nrows, D_MODEL), lambda: (0, 0)),
        ],
        out_specs=pl.BlockSpec(memory_space=pl.ANY),
        out_shape=jax.ShapeDtypeStruct((batch, rows4, lanes), jnp.float32),
        scratch_shapes=[
            pltpu.VMEM((rows4, lanes), jnp.float32),
            pltpu.SemaphoreType.DMA,
        ],
    )(jnp.asarray(onehot), jnp.asarray(table))
    return out2d.reshape(batch, seq, D_MODEL)


# speedup vs baseline: 6.0152x; 6.0152x over previous
import math

import jax
import jax.numpy as jnp
import numpy as np
from jax.experimental import pallas as pl
from jax.experimental.pallas import tpu as pltpu

D_MODEL = 512
MAX_LEN = 512
CH = 64


def _pe_table_ext() -> np.ndarray:
    pe = np.zeros((MAX_LEN, D_MODEL), dtype=np.float32)
    position = np.arange(0, MAX_LEN, dtype=np.float32)[:, None]
    div_term = np.exp(
        np.arange(0, D_MODEL, 2, dtype=np.float32) * -(math.log(10000.0) / D_MODEL)
    )
    pe[:, 0::2] = np.sin(position * div_term)
    pe[:, 1::2] = np.cos(position * div_term)
    return np.concatenate([np.zeros((1, D_MODEL), np.float32), pe], axis=0)


def _gather_indices(t_lens, D) -> np.ndarray:
    parts = []
    for t in t_lens:
        parts.append(np.zeros((1,), np.int32))
        parts.append(np.linspace(0, D - 1, t).astype(np.int32) + 1)
    return np.concatenate(parts)


def kernel(modal_feat_0, modal_feat_1, modal_feat_2):
    modal_feats = (modal_feat_0, modal_feat_1, modal_feat_2)
    batch = modal_feats[0].shape[0]
    D = modal_feats[0].shape[1] - 1
    t_lens = [m.shape[1] - 1 for m in modal_feats]
    seq = sum(t_lens) + len(t_lens)

    nfull = seq // CH
    tail = seq - nfull * CH
    rows_pad = (nfull + 1) * CH

    table = _pe_table_ext()
    idx = _gather_indices(t_lens, D)
    nrows = table.shape[0]
    onehot = np.zeros((rows_pad, nrows), np.float32)
    onehot[np.arange(seq), idx] = 1.0

    def body(oh_ref, tab_ref, o_ref, temp, buf0, buf1, tailbuf, sem0, sem1, semt):
        temp[...] = jnp.dot(
            oh_ref[...], tab_ref[...], preferred_element_type=jnp.float32
        )
        bufs, sems = (buf0, buf1), (sem0, sem1)
        copies = []
        for c in range(nfull):
            i = c % 2
            if c >= 2:
                copies[c - 2].wait()
            bufs[i][...] = jnp.broadcast_to(
                temp[pl.ds(c * CH, CH)][:, None, :], (CH, batch, D_MODEL)
            )
            cp = pltpu.make_async_copy(
                bufs[i], o_ref.at[pl.ds(c * CH, CH)], sems[i]
            )
            cp.start()
            copies.append(cp)
        tailbuf[...] = jnp.broadcast_to(
            temp[pl.ds(nfull * CH, tail)][:, None, :], (tail, batch, D_MODEL)
        )
        cpt = pltpu.make_async_copy(tailbuf, o_ref.at[pl.ds(nfull * CH, tail)], semt)
        cpt.start()
        copies[-2].wait()
        copies[-1].wait()
        cpt.wait()

    out = pl.pallas_call(
        body,
        in_specs=[
            pl.BlockSpec((rows_pad, nrows), lambda: (0, 0)),
            pl.BlockSpec((nrows, D_MODEL), lambda: (0, 0)),
        ],
        out_specs=pl.BlockSpec(memory_space=pl.ANY),
        out_shape=jax.ShapeDtypeStruct((seq, batch, D_MODEL), jnp.float32),
        scratch_shapes=[
            pltpu.VMEM((rows_pad, D_MODEL), jnp.float32),
            pltpu.VMEM((CH, batch, D_MODEL), jnp.float32),
            pltpu.VMEM((CH, batch, D_MODEL), jnp.float32),
            pltpu.VMEM((tail, batch, D_MODEL), jnp.float32),
            pltpu.SemaphoreType.DMA,
            pltpu.SemaphoreType.DMA,
            pltpu.SemaphoreType.DMA,
        ],
    )(jnp.asarray(onehot), jnp.asarray(table))
    return jnp.transpose(out, (1, 0, 2))


# speedup vs baseline: 6.1533x; 1.0230x over previous
import math

import jax
import jax.numpy as jnp
import numpy as np
from jax.experimental import pallas as pl
from jax.experimental.pallas import tpu as pltpu

D_MODEL = 512
MAX_LEN = 512
CH = 64


def _pe_table_ext() -> np.ndarray:
    pe = np.zeros((MAX_LEN, D_MODEL), dtype=np.float32)
    position = np.arange(0, MAX_LEN, dtype=np.float32)[:, None]
    div_term = np.exp(
        np.arange(0, D_MODEL, 2, dtype=np.float32) * -(math.log(10000.0) / D_MODEL)
    )
    pe[:, 0::2] = np.sin(position * div_term)
    pe[:, 1::2] = np.cos(position * div_term)
    return np.concatenate([np.zeros((1, D_MODEL), np.float32), pe], axis=0)


def _gather_indices(t_lens, D) -> np.ndarray:
    parts = []
    for t in t_lens:
        parts.append(np.zeros((1,), np.int32))
        parts.append(np.linspace(0, D - 1, t).astype(np.int32) + 1)
    return np.concatenate(parts)


def kernel(modal_feat_0, modal_feat_1, modal_feat_2):
    modal_feats = (modal_feat_0, modal_feat_1, modal_feat_2)
    batch = modal_feats[0].shape[0]
    D = modal_feats[0].shape[1] - 1
    t_lens = [m.shape[1] - 1 for m in modal_feats]
    seq = sum(t_lens) + len(t_lens)

    nfull = seq // CH
    tail = seq - nfull * CH
    rows_pad = (nfull + 1) * CH

    table = _pe_table_ext()
    idx = _gather_indices(t_lens, D)
    nrows = table.shape[0]
    onehot = np.zeros((rows_pad, nrows), np.float32)
    onehot[np.arange(seq), idx] = 1.0

    def body(oh_ref, tab_ref, o_ref, buf0, buf1, tailbuf, sem0, sem1, semt):
        bufs, sems = (buf0, buf1), (sem0, sem1)
        copies = []
        for c in range(nfull):
            i = c % 2
            if c >= 2:
                copies[c - 2].wait()
            rows = jnp.dot(
                oh_ref[pl.ds(c * CH, CH)],
                tab_ref[...],
                preferred_element_type=jnp.float32,
            )
            bufs[i][...] = jnp.broadcast_to(
                rows[:, None, :], (CH, batch, D_MODEL)
            )
            cp = pltpu.make_async_copy(
                bufs[i], o_ref.at[pl.ds(c * CH, CH)], sems[i]
            )
            cp.start()
            copies.append(cp)
        trows = jnp.dot(
            oh_ref[pl.ds(nfull * CH, tail)],
            tab_ref[...],
            preferred_element_type=jnp.float32,
        )
        tailbuf[...] = jnp.broadcast_to(trows[:, None, :], (tail, batch, D_MODEL))
        cpt = pltpu.make_async_copy(tailbuf, o_ref.at[pl.ds(nfull * CH, tail)], semt)
        cpt.start()
        copies[-2].wait()
        copies[-1].wait()
        cpt.wait()

    out = pl.pallas_call(
        body,
        in_specs=[
            pl.BlockSpec((rows_pad, nrows), lambda: (0, 0)),
            pl.BlockSpec((nrows, D_MODEL), lambda: (0, 0)),
        ],
        out_specs=pl.BlockSpec(memory_space=pl.ANY),
        out_shape=jax.ShapeDtypeStruct((seq, batch, D_MODEL), jnp.float32),
        scratch_shapes=[
            pltpu.VMEM((CH, batch, D_MODEL), jnp.float32),
            pltpu.VMEM((CH, batch, D_MODEL), jnp.float32),
            pltpu.VMEM((tail, batch, D_MODEL), jnp.float32),
            pltpu.SemaphoreType.DMA,
            pltpu.SemaphoreType.DMA,
            pltpu.SemaphoreType.DMA,
        ],
    )(jnp.asarray(onehot), jnp.asarray(table))
    return jnp.transpose(out, (1, 0, 2))
